# baseline (speedup 1.0000x reference)
import jax
import jax.numpy as jnp
from jax.experimental import pallas as pl


def _body(tab_ref, out_ref):
    out_ref[...] = jnp.broadcast_to(tab_ref[...][None, :, :], out_ref.shape)


def kernel(batch_ref, table):
    B = batch_ref.shape[0]
    Q, H = table.shape
    BB = 8
    return pl.pallas_call(
        _body,
        grid=(B // BB,),
        in_specs=[pl.BlockSpec((Q, H), lambda i: (0, 0))],
        out_specs=pl.BlockSpec((BB, Q, H), lambda i: (i, 0, 0)),
        out_shape=jax.ShapeDtypeStruct((B, Q, H), table.dtype),
    )(table)


# speedup vs baseline: 1.0274x; 1.0274x over previous
import jax
import jax.numpy as jnp
from jax.experimental import pallas as pl
from jax.experimental.pallas import tpu as pltpu


def _body(tab_ref, out_hbm, sem):
    B = out_hbm.shape[0]
    for b in range(B):
        pltpu.make_async_copy(tab_ref, out_hbm.at[b], sem).start()
    for b in range(B):
        pltpu.make_async_copy(tab_ref, out_hbm.at[b], sem).wait()


def kernel(batch_ref, table):
    B = batch_ref.shape[0]
    Q, H = table.shape
    return pl.pallas_call(
        _body,
        in_specs=[pl.BlockSpec((Q, H), lambda: (0, 0))],
        out_specs=pl.BlockSpec(memory_space=pl.ANY),
        out_shape=jax.ShapeDtypeStruct((B, Q, H), table.dtype),
        scratch_shapes=[pltpu.SemaphoreType.DMA],
    )(table)
